# baseline (speedup 1.0000x reference)
import math

import jax
import jax.numpy as jnp
from jax import lax
from jax.experimental import pallas as pl
from jax.experimental.pallas import tpu as pltpu
from jax.experimental.pallas import tpu_sc as plsc

_BASE = 2
_PAT_ROWS = 32


def _sc_body(emb_hbm, out_hbm, pat_a, pat_b, idx_a, idx_b, sem_a, sem_b,
             gsem_a, gsem_b):
    cid = lax.axis_index("c")
    sid = lax.axis_index("s")
    wid = sid * 2 + cid
    max_level, t_total, _ = out_hbm.shape
    chunk = t_total // 32
    t0 = wid * chunk
    rows = _PAT_ROWS
    nstream = chunk // rows

    pats = (pat_a, pat_b)
    idxs = (idx_a, idx_b)
    sems = (sem_a, sem_b)
    gsems = (gsem_a, gsem_b)
    pending = {0: [], 1: []}
    lane = lax.iota(jnp.int32, 16)
    for l in range(max_level):
        slot = l % 2
        p, idx, sem, gsem = pats[slot], idxs[slot], sems[slot], gsems[slot]
        for h in pending[slot]:
            h.wait()
        pending[slot] = []
        stride = 1 if l % _BASE == 0 else 0
        vals = (_BASE * l) + (lane % _BASE) * stride
        for r0 in range(0, rows, 16):
            idx[pl.ds(r0, 16)] = vals
        pltpu.async_copy(emb_hbm.at[idx], p, gsem).wait()
        for k in range(nstream):
            h = pltpu.async_copy(
                p, out_hbm.at[l, pl.ds(t0 + k * rows, rows)], sem)
            pending[slot].append(h)
    for lst in pending.values():
        for h in lst:
            h.wait()


def kernel(x, emb):
    B, T = x.shape
    del B
    max_level = int(math.ceil(math.log(T, _BASE)))
    n_emb, d = emb.shape
    del n_emb

    mesh = plsc.VectorSubcoreMesh(core_axis_name="c", subcore_axis_name="s")
    k = pl.kernel(
        _sc_body,
        out_type=jax.ShapeDtypeStruct((max_level, T, d), emb.dtype),
        mesh=mesh,
        scratch_types=[
            pltpu.VMEM((_PAT_ROWS, d), emb.dtype),
            pltpu.VMEM((_PAT_ROWS, d), emb.dtype),
            pltpu.VMEM((_PAT_ROWS,), jnp.int32),
            pltpu.VMEM((_PAT_ROWS,), jnp.int32),
            pltpu.SemaphoreType.DMA,
            pltpu.SemaphoreType.DMA,
            pltpu.SemaphoreType.DMA,
            pltpu.SemaphoreType.DMA,
        ],
    )
    return k(emb)


# speedup vs baseline: 2.7633x; 2.7633x over previous
import math

import jax
import jax.numpy as jnp
from jax import lax
from jax.experimental import pallas as pl
from jax.experimental.pallas import tpu as pltpu
from jax.experimental.pallas import tpu_sc as plsc

_BASE = 2
_REP = 8


def _sc_body(emb_hbm, out_hbm, pat, idx, gsem, sem):
    cid = lax.axis_index("c")
    sid = lax.axis_index("s")
    wid = sid * 2 + cid
    max_level, t_total, _ = out_hbm.shape
    chunk = t_total // 32
    t0 = wid * chunk
    nstream = chunk // _REP
    npad = idx.shape[0]

    for c0 in range(0, npad, 16):
        j = c0 + lax.iota(jnp.int32, 16)
        lvl = j >> 3
        par = j & 1
        vals = (lvl << 1) + par * (1 - (lvl & 1))
        vals = jnp.where(lvl < max_level, vals, 0)
        idx[pl.ds(c0, 16)] = vals
    pltpu.async_copy(emb_hbm.at[idx], pat, gsem).wait()

    pending = []
    for l in range(max_level):
        src = pat.at[pl.ds(l * _REP, _REP)]
        for k in range(nstream):
            h = pltpu.async_copy(
                src, out_hbm.at[l, pl.ds(t0 + k * _REP, _REP)], sem)
            pending.append(h)
    for h in pending:
        h.wait()


def kernel(x, emb):
    B, T = x.shape
    del B
    max_level = int(math.ceil(math.log(T, _BASE)))
    d = emb.shape[1]
    npad = -(-max_level * _REP // 16) * 16

    mesh = plsc.VectorSubcoreMesh(core_axis_name="c", subcore_axis_name="s")
    k = pl.kernel(
        _sc_body,
        out_type=jax.ShapeDtypeStruct((max_level, T, d), emb.dtype),
        mesh=mesh,
        scratch_types=[
            pltpu.VMEM((npad, d), emb.dtype),
            pltpu.VMEM((npad,), jnp.int32),
            pltpu.SemaphoreType.DMA,
            pltpu.SemaphoreType.DMA,
        ],
    )
    return k(emb)
